# baseline (speedup 1.0000x reference)
import functools

import jax
import jax.numpy as jnp
from jax import lax
from jax.experimental import pallas as pl
from jax.experimental.pallas import tpu as pltpu
from jax.experimental.pallas import tpu_sc as plsc

_B, _L, _D = 64, 1024, 128
_SG = 8
_LG = 256
_V = 32
_NU = _LG // _V
_NBUF = 3
_LANES = 16


@jax.jit
def _sc_embed(x, token_table, pos_table):
  mesh = plsc.VectorSubcoreMesh(core_axis_name="c", subcore_axis_name="s")

  @functools.partial(
      pl.kernel,
      mesh=mesh,
      out_type=jax.ShapeDtypeStruct((_B, _L, _D), jnp.float32),
      scratch_types=[
          pltpu.VMEM((_SG, _LG), jnp.int32),
          pltpu.VMEM((_NBUF, _SG, _V, _D), jnp.float32),
          pltpu.VMEM((_NBUF, _V, _D), jnp.float32),
          pltpu.SemaphoreType.DMA((_NBUF,)),
          pltpu.SemaphoreType.DMA((_NBUF,)),
          pltpu.SemaphoreType.DMA((_NBUF,)),
      ],
  )
  def k(x_hbm, tok_hbm, pos_hbm, out_hbm, idx_v, rows, posb, gsem, psem, osem):
    wid = lax.axis_index("s") * 2 + lax.axis_index("c")
    g0row = (wid // 4) * _SG
    l0 = (wid % 4) * _LG

    pltpu.sync_copy(x_hbm.at[pl.ds(g0row, _SG), pl.ds(l0, _LG)], idx_v)

    def start(v):
      b = lax.rem(v, _NBUF)
      for s in range(_SG):
        pltpu.async_copy(
            tok_hbm.at[idx_v.at[s, pl.ds(v * _V, _V)]],
            rows.at[b, s], gsem.at[b])
      pltpu.async_copy(
          pos_hbm.at[pl.ds(l0 + v * _V, _V), :], posb.at[b], psem.at[b])

    def wait_gathers(v):
      b = lax.rem(v, _NBUF)
      for s in range(_SG):
        pltpu.make_async_copy(
            tok_hbm.at[idx_v.at[s, pl.ds(0, _V)]],
            rows.at[b, s], gsem.at[b]).wait()
      pltpu.make_async_copy(
          pos_hbm.at[pl.ds(l0, _V), :], posb.at[b], psem.at[b]).wait()

    def issue_writes(v):
      b = lax.rem(v, _NBUF)
      for s in range(_SG):
        pltpu.async_copy(
            rows.at[b, s],
            out_hbm.at[g0row + s, pl.ds(l0 + v * _V, _V), :], osem.at[b])

    def wait_writes(v):
      b = lax.rem(v, _NBUF)
      for s in range(_SG):
        pltpu.make_async_copy(
            rows.at[b, s],
            out_hbm.at[g0row + s, pl.ds(l0, _V), :], osem.at[b]).wait()

    start(jnp.int32(0))
    start(jnp.int32(1))

    def unit_body(v, _):
      b = lax.rem(v, _NBUF)

      @pl.when(v + 2 < _NU)
      def _():
        @pl.when(v >= 1)
        def _():
          wait_writes(v - 1)
        start(v + 2)

      wait_gathers(v)

      def add_body(i, _):
        for j in range(_D // _LANES):
          o = j * _LANES
          p = posb[b, i, pl.ds(o, _LANES)]
          for s in range(_SG):
            rows[b, s, i, pl.ds(o, _LANES)] = (
                rows[b, s, i, pl.ds(o, _LANES)] + p)
        return 0

      lax.fori_loop(0, _V, add_body, 0)
      issue_writes(v)
      return 0

    lax.fori_loop(0, _NU, unit_body, jnp.int32(0))
    for v in range(_NU - 3, _NU):
      wait_writes(jnp.int32(v))

  return k(x, token_table, pos_table)


def kernel(x, token_table, pos_table):
  return _sc_embed(x, token_table, pos_table)


# speedup vs baseline: 1.5741x; 1.5741x over previous
import functools

import jax
import jax.numpy as jnp
from jax import lax
from jax.experimental import pallas as pl
from jax.experimental.pallas import tpu as pltpu
from jax.experimental.pallas import tpu_sc as plsc

_B, _L, _D = 64, 1024, 128
_SG = 8
_LG = 256
_V = 32
_NU = _LG // _V
_NBUF = 3
_LANES = 16


@jax.jit
def _sc_embed(x, token_table, pos_table):
  mesh = plsc.VectorSubcoreMesh(core_axis_name="c", subcore_axis_name="s")

  @functools.partial(
      pl.kernel,
      mesh=mesh,
      out_type=jax.ShapeDtypeStruct((_B, _L, _D), jnp.float32),
      scratch_types=[
          pltpu.VMEM((_SG, _LG), jnp.int32),
          pltpu.VMEM((_NBUF, _SG, _V, _D), jnp.float32),
          pltpu.VMEM((_NBUF, _V, _D), jnp.float32),
          pltpu.SemaphoreType.DMA((_NBUF,)),
          pltpu.SemaphoreType.DMA((_NBUF,)),
          pltpu.SemaphoreType.DMA((_NBUF,)),
      ],
  )
  def k(x_hbm, tok_hbm, pos_hbm, out_hbm, idx_v, rows, posb, gsem, psem, osem):
    wid = lax.axis_index("s") * 2 + lax.axis_index("c")
    g0row = (wid // 4) * _SG
    l0 = (wid % 4) * _LG

    pltpu.sync_copy(x_hbm.at[pl.ds(g0row, _SG), pl.ds(l0, _LG)], idx_v)

    def start(v):
      b = v % _NBUF
      hs = [
          pltpu.async_copy(
              tok_hbm.at[idx_v.at[s, pl.ds(v * _V, _V)]],
              rows.at[b, s], gsem.at[b])
          for s in range(_SG)
      ]
      hs.append(pltpu.async_copy(
          pos_hbm.at[pl.ds(l0 + v * _V, _V), :], posb.at[b], psem.at[b]))
      return hs

    in_flight = {0: start(0), 1: start(1)}
    out_flight = {}
    for v in range(_NU):
      b = v % _NBUF
      if v + 2 < _NU:
        if v - 1 in out_flight:
          for h in out_flight.pop(v - 1):
            h.wait()
        in_flight[v + 2] = start(v + 2)
      for h in in_flight.pop(v):
        h.wait()

      def add_body(i, _):
        for j in range(_D // _LANES):
          o = j * _LANES
          p = posb[b, i, pl.ds(o, _LANES)]
          for s in range(_SG):
            rows[b, s, i, pl.ds(o, _LANES)] = (
                rows[b, s, i, pl.ds(o, _LANES)] + p)
        return 0

      lax.fori_loop(0, _V, add_body, 0)

      out_flight[v] = [
          pltpu.async_copy(
              rows.at[b, s],
              out_hbm.at[g0row + s, pl.ds(l0 + v * _V, _V), :], osem.at[b])
          for s in range(_SG)
      ]

    for v in sorted(out_flight):
      for h in out_flight[v]:
        h.wait()

  return k(x, token_table, pos_table)


def kernel(x, token_table, pos_table):
  return _sc_embed(x, token_table, pos_table)


# speedup vs baseline: 1.5952x; 1.0134x over previous
import functools

import jax
import jax.numpy as jnp
from jax import lax
from jax.experimental import pallas as pl
from jax.experimental.pallas import tpu as pltpu
from jax.experimental.pallas import tpu_sc as plsc

_B, _L, _D = 64, 1024, 128
_SG = 8
_LG = 256
_V = 32
_NU = _LG // _V
_NBUF = 3
_LANES = 16


@jax.jit
def _sc_embed(x, token_table, pos_table):
  mesh = plsc.VectorSubcoreMesh(core_axis_name="c", subcore_axis_name="s")

  @functools.partial(
      pl.kernel,
      mesh=mesh,
      out_type=jax.ShapeDtypeStruct((_B, _L, _D), jnp.float32),
      scratch_types=[
          pltpu.VMEM((_SG, _LG), jnp.int32),
          pltpu.VMEM((_NBUF, _SG, _V, _D), jnp.float32),
          pltpu.VMEM((_NBUF, _V, _D), jnp.float32),
          pltpu.SemaphoreType.DMA((_NBUF,)),
          pltpu.SemaphoreType.DMA((_NBUF,)),
          pltpu.SemaphoreType.DMA((_NBUF,)),
      ],
  )
  def k(x_hbm, tok_hbm, pos_hbm, out_hbm, idx_v, rows, posb, gsem, psem, osem):
    wid = lax.axis_index("s") * 2 + lax.axis_index("c")
    g0row = (wid // 4) * _SG
    l0 = (wid % 4) * _LG

    pltpu.sync_copy(x_hbm.at[pl.ds(g0row, _SG), pl.ds(l0, _LG)], idx_v)

    def start(v):
      b = v % _NBUF
      hs = [
          pltpu.async_copy(
              tok_hbm.at[idx_v.at[s, pl.ds(v * _V, _V)]],
              rows.at[b, s], gsem.at[b])
          for s in range(_SG)
      ]
      hs.append(pltpu.async_copy(
          pos_hbm.at[pl.ds(l0 + v * _V, _V), :], posb.at[b], psem.at[b]))
      return hs

    in_flight = {0: start(0), 1: start(1)}
    out_flight = {}
    for v in range(_NU):
      b = v % _NBUF
      if v + 2 < _NU:
        if v - 1 in out_flight:
          for h in out_flight.pop(v - 1):
            h.wait()
        in_flight[v + 2] = start(v + 2)
      for h in in_flight.pop(v):
        h.wait()

      @plsc.parallel_loop(0, _V * (_D // _LANES), unroll=4)
      def _add(t):
        i = t % _V
        o = (t // _V) * _LANES
        p = posb[b, i, pl.ds(o, _LANES)]
        for s in range(_SG):
          rows[b, s, i, pl.ds(o, _LANES)] = (
              rows[b, s, i, pl.ds(o, _LANES)] + p)

      out_flight[v] = [
          pltpu.async_copy(
              rows.at[b, s],
              out_hbm.at[g0row + s, pl.ds(l0 + v * _V, _V), :], osem.at[b])
          for s in range(_SG)
      ]

    for v in sorted(out_flight):
      for h in out_flight[v]:
        h.wait()

  return k(x, token_table, pos_table)


def kernel(x, token_table, pos_table):
  return _sc_embed(x, token_table, pos_table)


# speedup vs baseline: 1.6056x; 1.0065x over previous
import functools

import jax
import jax.numpy as jnp
from jax import lax
from jax.experimental import pallas as pl
from jax.experimental.pallas import tpu as pltpu
from jax.experimental.pallas import tpu_sc as plsc

_B, _L, _D = 64, 1024, 128
_SG = 8
_LG = 256
_V = 32
_NU = _LG // _V
_NBUF = 3
_LANES = 16


@jax.jit
def _sc_embed(x, token_table, pos_table):
  mesh = plsc.VectorSubcoreMesh(core_axis_name="c", subcore_axis_name="s")

  @functools.partial(
      pl.kernel,
      mesh=mesh,
      out_type=jax.ShapeDtypeStruct((_B, _L, _D), jnp.float32),
      scratch_types=[
          pltpu.VMEM((_SG, _LG), jnp.int32),
          pltpu.VMEM((_NBUF, _SG, _V, _D), jnp.float32),
          pltpu.VMEM((_NBUF, _V, _D), jnp.float32),
          pltpu.SemaphoreType.DMA((_NBUF,)),
          pltpu.SemaphoreType.DMA((_NBUF,)),
          pltpu.SemaphoreType.DMA((_NBUF,)),
      ],
  )
  def k(x_hbm, tok_hbm, pos_hbm, out_hbm, idx_v, rows, posb, gsem, psem, osem):
    wid = lax.axis_index("s") * 2 + lax.axis_index("c")
    g0row = (wid // 4) * _SG
    l0 = (wid % 4) * _LG

    pltpu.sync_copy(x_hbm.at[pl.ds(g0row, _SG), pl.ds(l0, _LG)], idx_v)

    def start(v):
      b = v % _NBUF
      hs = [
          pltpu.async_copy(
              tok_hbm.at[idx_v.at[s, pl.ds(v * _V, _V)]],
              rows.at[b, s], gsem.at[b])
          for s in range(_SG)
      ]
      hs.append(pltpu.async_copy(
          pos_hbm.at[pl.ds(l0 + v * _V, _V), :], posb.at[b], psem.at[b]))
      return hs

    in_flight = {0: start(0), 1: start(1)}
    out_flight = {}
    for v in range(_NU):
      b = v % _NBUF
      if v + 2 < _NU:
        if v - 1 in out_flight:
          for h in out_flight.pop(v - 1):
            h.wait()
        in_flight[v + 2] = start(v + 2)
      for h in in_flight.pop(v):
        h.wait()

      @plsc.parallel_loop(0, _V * (_D // _LANES), unroll=2)
      def _add(t):
        i = t % _V
        o = (t // _V) * _LANES
        p = posb[b, i, pl.ds(o, _LANES)]
        for s in range(_SG):
          rows[b, s, i, pl.ds(o, _LANES)] = (
              rows[b, s, i, pl.ds(o, _LANES)] + p)

      out_flight[v] = [
          pltpu.async_copy(
              rows.at[b, s],
              out_hbm.at[g0row + s, pl.ds(l0 + v * _V, _V), :], osem.at[b])
          for s in range(_SG)
      ]

    for v in sorted(out_flight):
      for h in out_flight[v]:
        h.wait()

  return k(x, token_table, pos_table)


def kernel(x, token_table, pos_table):
  return _sc_embed(x, token_table, pos_table)


# speedup vs baseline: 1.6434x; 1.0236x over previous
import functools

import jax
import jax.numpy as jnp
from jax import lax
from jax.experimental import pallas as pl
from jax.experimental.pallas import tpu as pltpu
from jax.experimental.pallas import tpu_sc as plsc

_B, _L, _D = 64, 1024, 128
_SG = 8
_LG = 256
_V = 32
_NU = _LG // _V
_NBUF = 3
_LANES = 16


@jax.jit
def _sc_embed(x, token_table, pos_table):
  mesh = plsc.VectorSubcoreMesh(core_axis_name="c", subcore_axis_name="s")

  @functools.partial(
      pl.kernel,
      mesh=mesh,
      out_type=jax.ShapeDtypeStruct((_B, _L, _D), jnp.float32),
      scratch_types=[
          pltpu.VMEM((_SG, _LG), jnp.int32),
          pltpu.VMEM((_NBUF, _SG, _V, _D), jnp.float32),
          pltpu.VMEM((_NBUF, _V, _D), jnp.float32),
          pltpu.SemaphoreType.DMA((_NBUF,)),
          pltpu.SemaphoreType.DMA((_NBUF,)),
          pltpu.SemaphoreType.DMA((_NBUF,)),
      ],
  )
  def k(x_hbm, tok_hbm, pos_hbm, out_hbm, idx_v, rows, posb, gsem, psem, osem):
    wid = lax.axis_index("s") * 2 + lax.axis_index("c")
    g0row = (wid // 4) * _SG
    l0 = (wid % 4) * _LG

    pltpu.sync_copy(x_hbm.at[pl.ds(g0row, _SG), pl.ds(l0, _LG)], idx_v)

    def start(v):
      b = v % _NBUF
      hs = [
          pltpu.async_copy(
              tok_hbm.at[idx_v.at[s, pl.ds(v * _V, _V)]],
              rows.at[b, s], gsem.at[b])
          for s in range(_SG)
      ]
      hs.append(pltpu.async_copy(
          pos_hbm.at[pl.ds(l0 + v * _V, _V), :], posb.at[b], psem.at[b]))
      return hs

    in_flight = {0: start(0), 1: start(1)}
    out_flight = {}
    for v in range(_NU):
      b = v % _NBUF
      if v + 2 < _NU:
        if v - 1 in out_flight:
          for h in out_flight.pop(v - 1):
            h.wait()
        in_flight[v + 2] = start(v + 2)
      for h in in_flight.pop(v):
        h.wait()

      @plsc.parallel_loop(0, _V * (_D // _LANES), unroll=2)
      def _add(t):
        i = t % _V
        o = (t // _V) * _LANES
        p = posb[b, i, pl.ds(o, _LANES)]
        for s in range(_SG):
          rows[b, s, i, pl.ds(o, _LANES)] = (
              rows[b, s, i, pl.ds(o, _LANES)] + p)

      out_flight[v] = [
          pltpu.async_copy(
              rows.at[b],
              out_hbm.at[pl.ds(g0row, _SG), pl.ds(l0 + v * _V, _V), :],
              osem.at[b])
      ]

    for v in sorted(out_flight):
      for h in out_flight[v]:
        h.wait()

  return k(x, token_table, pos_table)


def kernel(x, token_table, pos_table):
  return _sc_embed(x, token_table, pos_table)
